# baseline (speedup 1.0000x reference)
import functools

import jax
import jax.numpy as jnp
from jax import lax
from jax.experimental import pallas as pl
from jax.experimental.pallas import tpu as pltpu
from jax.experimental.pallas import tpu_sc as plsc

SAMPLE_N = 512
KNN = 16
LANES = 16
NCORES = 2
NSUB = 16
NW = NCORES * NSUB

BATCH = 8
NPTS = 8192
NCH = 256

ROWS_W = (BATCH * SAMPLE_N) // NW
NCHUNK = NPTS // LANES
CAND_MAX = NPTS + LANES
W_PER_B = NW // BATCH
CH_W = NCH // W_PER_B
CH_BLK = 8


def _sc_mesh():
    return plsc.VectorSubcoreMesh(
        core_axis_name="c", subcore_axis_name="s",
        num_cores=NCORES, num_subcores=NSUB,
    )


def _fps_body(cx_ref, cy_ref, cz_ref, sx_ref, sy_ref, sz_ref):
    cx = cx_ref[...]
    cy = cy_ref[...]
    cz = cz_ref[...]
    B, N = cx.shape

    dx = cx - cx[:, 0:1]
    dy = cy - cy[:, 0:1]
    dz = cz - cz[:, 0:1]
    min_d = dx * dx + dy * dy + dz * dz

    lane_iota = jax.lax.broadcasted_iota(jnp.int32, (B, N), 1)
    col_iota = jax.lax.broadcasted_iota(jnp.int32, (B, SAMPLE_N), 1)

    sx0 = jnp.where(col_iota == 0, cx[:, 0:1], 0.0)
    sy0 = jnp.where(col_iota == 0, cy[:, 0:1], 0.0)
    sz0 = jnp.where(col_iota == 0, cz[:, 0:1], 0.0)

    def body(i, carry):
        min_d, sx, sy, sz = carry
        m = jnp.max(min_d, axis=1, keepdims=True)
        idx = jnp.min(
            jnp.where(min_d == m, lane_iota, jnp.int32(N)), axis=1, keepdims=True
        )
        sel = lane_iota == idx
        px = jnp.sum(jnp.where(sel, cx, 0.0), axis=1, keepdims=True)
        py = jnp.sum(jnp.where(sel, cy, 0.0), axis=1, keepdims=True)
        pz = jnp.sum(jnp.where(sel, cz, 0.0), axis=1, keepdims=True)
        ddx = cx - px
        ddy = cy - py
        ddz = cz - pz
        d = ddx * ddx + ddy * ddy + ddz * ddz
        min_d = jnp.minimum(min_d, d)
        upd = col_iota == i
        sx = jnp.where(upd, px, sx)
        sy = jnp.where(upd, py, sy)
        sz = jnp.where(upd, pz, sz)
        return (min_d, sx, sy, sz)

    _, sx, sy, sz = jax.lax.fori_loop(1, SAMPLE_N, body, (min_d, sx0, sy0, sz0))
    sx_ref[...] = sx
    sy_ref[...] = sy
    sz_ref[...] = sz


def _fps(cx, cy, cz):
    B = cx.shape[0]
    out = jax.ShapeDtypeStruct((B, SAMPLE_N), jnp.float32)
    return pl.pallas_call(_fps_body, out_shape=[out, out, out])(cx, cy, cz)


def _knn_body(cx_hbm, cy_hbm, cz_hbm, ssx_hbm, ssy_hbm, ssz_hbm, idx_hbm,
              cxv, cyv, czv, c2v, sxv, syv, szv, idxout):
    wid = lax.axis_index("s") * NCORES + lax.axis_index("c")
    b = wid // W_PER_B
    r0 = wid * ROWS_W

    pltpu.sync_copy(cx_hbm.at[b], cxv)
    pltpu.sync_copy(cy_hbm.at[b], cyv)
    pltpu.sync_copy(cz_hbm.at[b], czv)
    pltpu.sync_copy(ssx_hbm.at[pl.ds(r0 * LANES, ROWS_W * LANES)], sxv)
    pltpu.sync_copy(ssy_hbm.at[pl.ds(r0 * LANES, ROWS_W * LANES)], syv)
    pltpu.sync_copy(ssz_hbm.at[pl.ds(r0 * LANES, ROWS_W * LANES)], szv)

    iota = lax.iota(jnp.int32, LANES)
    inf16 = jnp.full((LANES,), jnp.inf, jnp.float32)
    zero16i = jnp.zeros((LANES,), jnp.int32)

    def pre(j, _):
        base = j * LANES
        x = cxv[pl.ds(base, LANES)]
        y = cyv[pl.ds(base, LANES)]
        z = czv[pl.ds(base, LANES)]
        c2v[pl.ds(base, LANES)] = (x * x + y * y) + z * z
        return 0

    lax.fori_loop(0, NCHUNK, pre, 0)

    def row_fn(r, _):
        sx = sxv[pl.ds(r * LANES, LANES)]
        sy = syv[pl.ds(r * LANES, LANES)]
        sz = szv[pl.ds(r * LANES, LANES)]

        def c1(j, carry):
            Rv, Ri, tau = carry
            base = j * LANES
            t = sx * cxv[pl.ds(base, LANES)]
            t = t + sy * cyv[pl.ds(base, LANES)]
            t = t + sz * czv[pl.ds(base, LANES)]
            sc = c2v[pl.ds(base, LANES)] - (t + t)
            pc = plsc.all_reduce_population_count(sc <= tau)

            def merge(args):
                Rv, Ri = args
                Cv, Ci = plsc.sort_key_val(sc, iota + base, descending=True)
                mm = Cv < Rv
                Ev = jnp.where(mm, Cv, Rv)
                Ei = jnp.where(mm, Ci, Ri)
                Sv, Si = plsc.sort_key_val(Ev, Ei)
                return Sv, Si, Sv[LANES - 1]

            def skip(args):
                Rv, Ri = args
                return Rv, Ri, tau

            return lax.cond(pc[0] > 0, merge, skip, (Rv, Ri))

        _, Ri, _ = lax.fori_loop(
            0, NCHUNK, c1, (inf16, zero16i, jnp.float32(jnp.inf))
        )
        idxout[pl.ds(r * LANES, LANES)] = Ri
        return 0

    lax.fori_loop(0, ROWS_W, row_fn, 0)
    pltpu.sync_copy(idxout, idx_hbm.at[pl.ds(r0 * LANES, ROWS_W * LANES)])


def _knn(cx, cy, cz, ssx, ssy, ssz):
    call = pl.kernel(
        _knn_body,
        out_type=jax.ShapeDtypeStruct((BATCH * SAMPLE_N * KNN,), jnp.int32),
        mesh=_sc_mesh(),
        compiler_params=pltpu.CompilerParams(needs_layout_passes=False),
        scratch_types=[
            pltpu.VMEM((NPTS,), jnp.float32),
            pltpu.VMEM((NPTS,), jnp.float32),
            pltpu.VMEM((NPTS,), jnp.float32),
            pltpu.VMEM((NPTS,), jnp.float32),
            pltpu.VMEM((ROWS_W * LANES,), jnp.float32),
            pltpu.VMEM((ROWS_W * LANES,), jnp.float32),
            pltpu.VMEM((ROWS_W * LANES,), jnp.float32),
            pltpu.VMEM((ROWS_W * LANES,), jnp.int32),
        ],
    )
    return call(cx, cy, cz, ssx, ssy, ssz)


def _pool_body(x_hbm, idxt_hbm, out_hbm, idxv, rowbuf, outbuf):
    wid = lax.axis_index("s") * NCORES + lax.axis_index("c")
    b = wid // W_PER_B
    c0 = (wid % W_PER_B) * CH_W

    pltpu.sync_copy(idxt_hbm.at[b], idxv)

    def cb_fn(cb, _):
        cbase = c0 + cb * CH_BLK
        pltpu.sync_copy(x_hbm.at[b, pl.ds(cbase, CH_BLK), :], rowbuf)

        def g_fn(g, _):
            s0 = g * LANES
            ivs = [idxv[pl.ds(k * SAMPLE_N + s0, LANES)] for k in range(KNN)]
            for cl in range(CH_BLK):
                row = jnp.full((LANES,), cl, jnp.int32)
                acc = plsc.load_gather(rowbuf, [row, ivs[0]])
                for k in range(1, KNN):
                    acc = jnp.maximum(acc, plsc.load_gather(rowbuf, [row, ivs[k]]))
                outbuf[cl, pl.ds(s0, LANES)] = acc
            return 0

        lax.fori_loop(0, SAMPLE_N // LANES, g_fn, 0)
        pltpu.sync_copy(outbuf, out_hbm.at[b, pl.ds(cbase, CH_BLK), :])
        return 0

    lax.fori_loop(0, CH_W // CH_BLK, cb_fn, 0)


def _pool(x, idxt):
    call = pl.kernel(
        _pool_body,
        out_type=jax.ShapeDtypeStruct((BATCH, NCH, SAMPLE_N), jnp.float32),
        mesh=_sc_mesh(),
        compiler_params=pltpu.CompilerParams(needs_layout_passes=False),
        scratch_types=[
            pltpu.VMEM((KNN * SAMPLE_N,), jnp.int32),
            pltpu.VMEM((CH_BLK, NPTS), jnp.float32),
            pltpu.VMEM((CH_BLK, SAMPLE_N), jnp.float32),
        ],
    )
    return call(x, idxt)


def kernel(x, coordinates):
    B, N, _ = coordinates.shape
    cx = coordinates[:, :, 0]
    cy = coordinates[:, :, 1]
    cz = coordinates[:, :, 2]
    sx, sy, sz = _fps(cx, cy, cz)

    ssx = jnp.broadcast_to(sx.reshape(-1)[:, None], (B * SAMPLE_N, LANES)).reshape(-1)
    ssy = jnp.broadcast_to(sy.reshape(-1)[:, None], (B * SAMPLE_N, LANES)).reshape(-1)
    ssz = jnp.broadcast_to(sz.reshape(-1)[:, None], (B * SAMPLE_N, LANES)).reshape(-1)

    knn_idx = _knn(cx, cy, cz, ssx, ssy, ssz)
    idxt = (
        knn_idx.reshape(B, SAMPLE_N, KNN)
        .transpose(0, 2, 1)
        .reshape(B, KNN * SAMPLE_N)
    )
    pooled_t = _pool(x, idxt)
    return pooled_t.transpose(0, 2, 1).reshape(B, -1)


# speedup vs baseline: 2.5114x; 2.5114x over previous
import functools

import jax
import jax.numpy as jnp
from jax import lax
from jax.experimental import pallas as pl
from jax.experimental.pallas import tpu as pltpu
from jax.experimental.pallas import tpu_sc as plsc

SAMPLE_N = 512
KNN = 16
LANES = 16
NCORES = 2
NSUB = 16
NW = NCORES * NSUB

BATCH = 8
NPTS = 8192
NCH = 256

ROWS_W = (BATCH * SAMPLE_N) // NW
NCHUNK = NPTS // LANES
CAND_MAX = NPTS + LANES
W_PER_B = NW // BATCH
CH_W = NCH // W_PER_B
CH_BLK = 8


def _sc_mesh():
    return plsc.VectorSubcoreMesh(
        core_axis_name="c", subcore_axis_name="s",
        num_cores=NCORES, num_subcores=NSUB,
    )


def _fps_body(cx_ref, cy_ref, cz_ref, sx_ref, sy_ref, sz_ref):
    cx = cx_ref[...]
    cy = cy_ref[...]
    cz = cz_ref[...]
    B, N = cx.shape

    dx = cx - cx[:, 0:1]
    dy = cy - cy[:, 0:1]
    dz = cz - cz[:, 0:1]
    min_d = dx * dx + dy * dy + dz * dz

    lane_iota = jax.lax.broadcasted_iota(jnp.int32, (B, N), 1)
    col_iota = jax.lax.broadcasted_iota(jnp.int32, (B, SAMPLE_N), 1)

    sx0 = jnp.where(col_iota == 0, cx[:, 0:1], 0.0)
    sy0 = jnp.where(col_iota == 0, cy[:, 0:1], 0.0)
    sz0 = jnp.where(col_iota == 0, cz[:, 0:1], 0.0)

    def body(i, carry):
        min_d, sx, sy, sz = carry
        m = jnp.max(min_d, axis=1, keepdims=True)
        idx = jnp.min(
            jnp.where(min_d == m, lane_iota, jnp.int32(N)), axis=1, keepdims=True
        )
        sel = lane_iota == idx
        px = jnp.sum(jnp.where(sel, cx, 0.0), axis=1, keepdims=True)
        py = jnp.sum(jnp.where(sel, cy, 0.0), axis=1, keepdims=True)
        pz = jnp.sum(jnp.where(sel, cz, 0.0), axis=1, keepdims=True)
        ddx = cx - px
        ddy = cy - py
        ddz = cz - pz
        d = ddx * ddx + ddy * ddy + ddz * ddz
        min_d = jnp.minimum(min_d, d)
        upd = col_iota == i
        sx = jnp.where(upd, px, sx)
        sy = jnp.where(upd, py, sy)
        sz = jnp.where(upd, pz, sz)
        return (min_d, sx, sy, sz)

    _, sx, sy, sz = jax.lax.fori_loop(1, SAMPLE_N, body, (min_d, sx0, sy0, sz0))
    sx_ref[...] = sx
    sy_ref[...] = sy
    sz_ref[...] = sz


def _fps(cx, cy, cz):
    B = cx.shape[0]
    out = jax.ShapeDtypeStruct((B, SAMPLE_N), jnp.float32)
    return pl.pallas_call(_fps_body, out_shape=[out, out, out])(cx, cy, cz)


def _knn_body(cx_hbm, cy_hbm, cz_hbm, ssx_hbm, ssy_hbm, ssz_hbm, idx_hbm,
              cxv, cyv, czv, sxv, syv, szv, d2buf, candv, candi, idxout):
    wid = lax.axis_index("s") * NCORES + lax.axis_index("c")
    b = wid // W_PER_B
    r0 = wid * ROWS_W

    pltpu.sync_copy(cx_hbm.at[b], cxv)
    pltpu.sync_copy(cy_hbm.at[b], cyv)
    pltpu.sync_copy(cz_hbm.at[b], czv)
    pltpu.sync_copy(ssx_hbm.at[pl.ds(r0 * LANES, ROWS_W * LANES)], sxv)
    pltpu.sync_copy(ssy_hbm.at[pl.ds(r0 * LANES, ROWS_W * LANES)], syv)
    pltpu.sync_copy(ssz_hbm.at[pl.ds(r0 * LANES, ROWS_W * LANES)], szv)

    iota = lax.iota(jnp.int32, LANES)
    inf16 = jnp.full((LANES,), jnp.inf, jnp.float32)
    zero16i = jnp.zeros((LANES,), jnp.int32)

    def row_fn(r, _):
        sx = sxv[pl.ds(r * LANES, LANES)]
        sy = syv[pl.ds(r * LANES, LANES)]
        sz = szv[pl.ds(r * LANES, LANES)]

        @plsc.parallel_loop(0, NPTS, step=LANES, unroll=4, carry=inf16)
        def M(base, Mc):
            dx = sx - cxv[pl.ds(base, LANES)]
            dy = sy - cyv[pl.ds(base, LANES)]
            dz = sz - czv[pl.ds(base, LANES)]
            d2 = (dx * dx + dy * dy) + dz * dz
            d2buf[pl.ds(base, LANES)] = d2
            return jnp.minimum(Mc, d2)
        Mv, _ = plsc.sort_key_val(M, iota, descending=True)
        tau = Mv[0]

        def c2(j, cnt):
            base = j * LANES
            d2c = d2buf[pl.ds(base, LANES)]
            m = d2c <= tau
            plsc.store_compressed(candv.at[pl.ds(cnt, LANES)], d2c, mask=m)
            plsc.store_compressed(candi.at[pl.ds(cnt, LANES)], iota + base, mask=m)
            pc = plsc.all_reduce_population_count(m)
            return cnt + pc[0]

        cnt = lax.fori_loop(0, NCHUNK, c2, jnp.int32(0))
        candv[pl.ds(cnt, LANES)] = inf16
        ng = (cnt + (LANES - 1)) // LANES

        def c3(g, carry):
            Rv, Ri = carry
            base = g * LANES
            Cv, Ci = plsc.sort_key_val(
                candv[pl.ds(base, LANES)], candi[pl.ds(base, LANES)],
                descending=True,
            )
            m = Cv < Rv
            Ev = jnp.where(m, Cv, Rv)
            Ei = jnp.where(m, Ci, Ri)
            Sv, Si = plsc.sort_key_val(Ev, Ei)
            return (Sv, Si)

        _, Ri = lax.fori_loop(0, ng, c3, (inf16, zero16i))
        idxout[pl.ds(r * LANES, LANES)] = Ri
        return 0

    lax.fori_loop(0, ROWS_W, row_fn, 0)
    pltpu.sync_copy(idxout, idx_hbm.at[pl.ds(r0 * LANES, ROWS_W * LANES)])


def _knn(cx, cy, cz, ssx, ssy, ssz):
    call = pl.kernel(
        _knn_body,
        out_type=jax.ShapeDtypeStruct((BATCH * SAMPLE_N * KNN,), jnp.int32),
        mesh=_sc_mesh(),
        compiler_params=pltpu.CompilerParams(needs_layout_passes=False),
        scratch_types=[
            pltpu.VMEM((NPTS,), jnp.float32),
            pltpu.VMEM((NPTS,), jnp.float32),
            pltpu.VMEM((NPTS,), jnp.float32),
            pltpu.VMEM((ROWS_W * LANES,), jnp.float32),
            pltpu.VMEM((ROWS_W * LANES,), jnp.float32),
            pltpu.VMEM((ROWS_W * LANES,), jnp.float32),
            pltpu.VMEM((NPTS,), jnp.float32),
            pltpu.VMEM((CAND_MAX,), jnp.float32),
            pltpu.VMEM((CAND_MAX,), jnp.int32),
            pltpu.VMEM((ROWS_W * LANES,), jnp.int32),
        ],
    )
    return call(cx, cy, cz, ssx, ssy, ssz)


def _pool_body(x_hbm, idxt_hbm, out_hbm, idxv, rowbuf, outbuf):
    wid = lax.axis_index("s") * NCORES + lax.axis_index("c")
    b = wid // W_PER_B
    c0 = (wid % W_PER_B) * CH_W

    pltpu.sync_copy(idxt_hbm.at[b], idxv)

    def cb_fn(cb, _):
        cbase = c0 + cb * CH_BLK
        pltpu.sync_copy(x_hbm.at[b, pl.ds(cbase, CH_BLK), :], rowbuf)

        def g_fn(g, _):
            s0 = g * LANES
            ivs = [idxv[pl.ds(k * SAMPLE_N + s0, LANES)] for k in range(KNN)]
            for cl in range(CH_BLK):
                row = jnp.full((LANES,), cl, jnp.int32)
                acc = plsc.load_gather(rowbuf, [row, ivs[0]])
                for k in range(1, KNN):
                    acc = jnp.maximum(acc, plsc.load_gather(rowbuf, [row, ivs[k]]))
                outbuf[cl, pl.ds(s0, LANES)] = acc
            return 0

        lax.fori_loop(0, SAMPLE_N // LANES, g_fn, 0)
        pltpu.sync_copy(outbuf, out_hbm.at[b, pl.ds(cbase, CH_BLK), :])
        return 0

    lax.fori_loop(0, CH_W // CH_BLK, cb_fn, 0)


def _pool(x, idxt):
    call = pl.kernel(
        _pool_body,
        out_type=jax.ShapeDtypeStruct((BATCH, NCH, SAMPLE_N), jnp.float32),
        mesh=_sc_mesh(),
        compiler_params=pltpu.CompilerParams(needs_layout_passes=False),
        scratch_types=[
            pltpu.VMEM((KNN * SAMPLE_N,), jnp.int32),
            pltpu.VMEM((CH_BLK, NPTS), jnp.float32),
            pltpu.VMEM((CH_BLK, SAMPLE_N), jnp.float32),
        ],
    )
    return call(x, idxt)


def kernel(x, coordinates):
    B, N, _ = coordinates.shape
    cx = coordinates[:, :, 0]
    cy = coordinates[:, :, 1]
    cz = coordinates[:, :, 2]
    sx, sy, sz = _fps(cx, cy, cz)

    ssx = jnp.broadcast_to(sx.reshape(-1)[:, None], (B * SAMPLE_N, LANES)).reshape(-1)
    ssy = jnp.broadcast_to(sy.reshape(-1)[:, None], (B * SAMPLE_N, LANES)).reshape(-1)
    ssz = jnp.broadcast_to(sz.reshape(-1)[:, None], (B * SAMPLE_N, LANES)).reshape(-1)

    knn_idx = _knn(cx, cy, cz, ssx, ssy, ssz)
    idxt = (
        knn_idx.reshape(B, SAMPLE_N, KNN)
        .transpose(0, 2, 1)
        .reshape(B, KNN * SAMPLE_N)
    )
    pooled_t = _pool(x, idxt)
    return pooled_t.transpose(0, 2, 1).reshape(B, -1)


# speedup vs baseline: 2.6948x; 1.0730x over previous
import functools

import jax
import jax.numpy as jnp
from jax import lax
from jax.experimental import pallas as pl
from jax.experimental.pallas import tpu as pltpu
from jax.experimental.pallas import tpu_sc as plsc

SAMPLE_N = 512
KNN = 16
LANES = 16
NCORES = 2
NSUB = 16
NW = NCORES * NSUB

BATCH = 8
NPTS = 8192
NCH = 256

ROWS_W = (BATCH * SAMPLE_N) // NW
NCHUNK = NPTS // LANES
CAND_MAX = NPTS + LANES
W_PER_B = NW // BATCH
CH_W = NCH // W_PER_B
CH_BLK = 8


def _sc_mesh():
    return plsc.VectorSubcoreMesh(
        core_axis_name="c", subcore_axis_name="s",
        num_cores=NCORES, num_subcores=NSUB,
    )


def _fps_body(cx_ref, cy_ref, cz_ref, sx_ref, sy_ref, sz_ref):
    cx = cx_ref[...]
    cy = cy_ref[...]
    cz = cz_ref[...]
    B, N = cx.shape

    dx = cx - cx[:, 0:1]
    dy = cy - cy[:, 0:1]
    dz = cz - cz[:, 0:1]
    min_d = dx * dx + dy * dy + dz * dz

    lane_iota = jax.lax.broadcasted_iota(jnp.int32, (B, N), 1)
    col_iota = jax.lax.broadcasted_iota(jnp.int32, (B, SAMPLE_N), 1)

    sx0 = jnp.where(col_iota == 0, cx[:, 0:1], 0.0)
    sy0 = jnp.where(col_iota == 0, cy[:, 0:1], 0.0)
    sz0 = jnp.where(col_iota == 0, cz[:, 0:1], 0.0)

    def body(i, carry):
        min_d, sx, sy, sz = carry
        m = jnp.max(min_d, axis=1, keepdims=True)
        idx = jnp.min(
            jnp.where(min_d == m, lane_iota, jnp.int32(N)), axis=1, keepdims=True
        )
        sel = lane_iota == idx
        px = jnp.sum(jnp.where(sel, cx, 0.0), axis=1, keepdims=True)
        py = jnp.sum(jnp.where(sel, cy, 0.0), axis=1, keepdims=True)
        pz = jnp.sum(jnp.where(sel, cz, 0.0), axis=1, keepdims=True)
        ddx = cx - px
        ddy = cy - py
        ddz = cz - pz
        d = ddx * ddx + ddy * ddy + ddz * ddz
        min_d = jnp.minimum(min_d, d)
        upd = col_iota == i
        sx = jnp.where(upd, px, sx)
        sy = jnp.where(upd, py, sy)
        sz = jnp.where(upd, pz, sz)
        return (min_d, sx, sy, sz)

    _, sx, sy, sz = jax.lax.fori_loop(1, SAMPLE_N, body, (min_d, sx0, sy0, sz0))
    sx_ref[...] = sx
    sy_ref[...] = sy
    sz_ref[...] = sz


def _fps(cx, cy, cz):
    B = cx.shape[0]
    out = jax.ShapeDtypeStruct((B, SAMPLE_N), jnp.float32)
    return pl.pallas_call(_fps_body, out_shape=[out, out, out])(cx, cy, cz)


def _knn_body(cx_hbm, cy_hbm, cz_hbm, ssx_hbm, ssy_hbm, ssz_hbm, idx_hbm,
              cxv, cyv, czv, sxv, syv, szv, d2buf, candv, candi, idxout):
    wid = lax.axis_index("s") * NCORES + lax.axis_index("c")
    b = wid // W_PER_B
    r0 = wid * ROWS_W

    pltpu.sync_copy(cx_hbm.at[b], cxv)
    pltpu.sync_copy(cy_hbm.at[b], cyv)
    pltpu.sync_copy(cz_hbm.at[b], czv)
    pltpu.sync_copy(ssx_hbm.at[pl.ds(r0 * LANES, ROWS_W * LANES)], sxv)
    pltpu.sync_copy(ssy_hbm.at[pl.ds(r0 * LANES, ROWS_W * LANES)], syv)
    pltpu.sync_copy(ssz_hbm.at[pl.ds(r0 * LANES, ROWS_W * LANES)], szv)

    iota = lax.iota(jnp.int32, LANES)
    inf16 = jnp.full((LANES,), jnp.inf, jnp.float32)
    zero16i = jnp.zeros((LANES,), jnp.int32)

    def row_fn(r, _):
        sx = sxv[pl.ds(r * LANES, LANES)]
        sy = syv[pl.ds(r * LANES, LANES)]
        sz = szv[pl.ds(r * LANES, LANES)]

        @plsc.parallel_loop(0, NPTS, step=LANES, unroll=4, carry=inf16)
        def M(base, Mc):
            dx = sx - cxv[pl.ds(base, LANES)]
            dy = sy - cyv[pl.ds(base, LANES)]
            dz = sz - czv[pl.ds(base, LANES)]
            d2 = (dx * dx + dy * dy) + dz * dz
            d2buf[pl.ds(base, LANES)] = d2
            return jnp.minimum(Mc, d2)
        Mv, _ = plsc.sort_key_val(M, iota, descending=True)
        tau = Mv[0]

        def c2(j, cnt):
            base = j * (LANES * 4)
            offs = cnt
            for u in range(4):
                ub = base + u * LANES
                d2c = d2buf[pl.ds(ub, LANES)]
                m = d2c <= tau
                plsc.store_compressed(candv.at[pl.ds(offs, LANES)], d2c, mask=m)
                plsc.store_compressed(candi.at[pl.ds(offs, LANES)], iota + ub, mask=m)
                pc = plsc.all_reduce_population_count(m)
                offs = offs + pc[0]
            return offs

        cnt = lax.fori_loop(0, NCHUNK // 4, c2, jnp.int32(0))
        candv[pl.ds(cnt, LANES)] = inf16
        ng = (cnt + (LANES - 1)) // LANES

        def c3(g, carry):
            Rv, Ri = carry
            base = g * LANES
            Cv, Ci = plsc.sort_key_val(
                candv[pl.ds(base, LANES)], candi[pl.ds(base, LANES)],
                descending=True,
            )
            m = Cv < Rv
            Ev = jnp.where(m, Cv, Rv)
            Ei = jnp.where(m, Ci, Ri)
            Sv, Si = plsc.sort_key_val(Ev, Ei)
            return (Sv, Si)

        _, Ri = lax.fori_loop(0, ng, c3, (inf16, zero16i))
        idxout[pl.ds(r * LANES, LANES)] = Ri
        return 0

    lax.fori_loop(0, ROWS_W, row_fn, 0)
    pltpu.sync_copy(idxout, idx_hbm.at[pl.ds(r0 * LANES, ROWS_W * LANES)])


def _knn(cx, cy, cz, ssx, ssy, ssz):
    call = pl.kernel(
        _knn_body,
        out_type=jax.ShapeDtypeStruct((BATCH * SAMPLE_N * KNN,), jnp.int32),
        mesh=_sc_mesh(),
        compiler_params=pltpu.CompilerParams(needs_layout_passes=False),
        scratch_types=[
            pltpu.VMEM((NPTS,), jnp.float32),
            pltpu.VMEM((NPTS,), jnp.float32),
            pltpu.VMEM((NPTS,), jnp.float32),
            pltpu.VMEM((ROWS_W * LANES,), jnp.float32),
            pltpu.VMEM((ROWS_W * LANES,), jnp.float32),
            pltpu.VMEM((ROWS_W * LANES,), jnp.float32),
            pltpu.VMEM((NPTS,), jnp.float32),
            pltpu.VMEM((CAND_MAX,), jnp.float32),
            pltpu.VMEM((CAND_MAX,), jnp.int32),
            pltpu.VMEM((ROWS_W * LANES,), jnp.int32),
        ],
    )
    return call(cx, cy, cz, ssx, ssy, ssz)


def _pool_body(x_hbm, idxt_hbm, out_hbm, idxv, rowbuf, outbuf):
    wid = lax.axis_index("s") * NCORES + lax.axis_index("c")
    b = wid // W_PER_B
    c0 = (wid % W_PER_B) * CH_W

    pltpu.sync_copy(idxt_hbm.at[b], idxv)

    def cb_fn(cb, _):
        cbase = c0 + cb * CH_BLK
        pltpu.sync_copy(x_hbm.at[b, pl.ds(cbase, CH_BLK), :], rowbuf)

        def g_fn(g, _):
            s0 = g * LANES
            ivs = [idxv[pl.ds(k * SAMPLE_N + s0, LANES)] for k in range(KNN)]
            for cl in range(CH_BLK):
                row = jnp.full((LANES,), cl, jnp.int32)
                acc = plsc.load_gather(rowbuf, [row, ivs[0]])
                for k in range(1, KNN):
                    acc = jnp.maximum(acc, plsc.load_gather(rowbuf, [row, ivs[k]]))
                outbuf[cl, pl.ds(s0, LANES)] = acc
            return 0

        lax.fori_loop(0, SAMPLE_N // LANES, g_fn, 0)
        pltpu.sync_copy(outbuf, out_hbm.at[b, pl.ds(cbase, CH_BLK), :])
        return 0

    lax.fori_loop(0, CH_W // CH_BLK, cb_fn, 0)


def _pool(x, idxt):
    call = pl.kernel(
        _pool_body,
        out_type=jax.ShapeDtypeStruct((BATCH, NCH, SAMPLE_N), jnp.float32),
        mesh=_sc_mesh(),
        compiler_params=pltpu.CompilerParams(needs_layout_passes=False),
        scratch_types=[
            pltpu.VMEM((KNN * SAMPLE_N,), jnp.int32),
            pltpu.VMEM((CH_BLK, NPTS), jnp.float32),
            pltpu.VMEM((CH_BLK, SAMPLE_N), jnp.float32),
        ],
    )
    return call(x, idxt)


def kernel(x, coordinates):
    B, N, _ = coordinates.shape
    cx = coordinates[:, :, 0]
    cy = coordinates[:, :, 1]
    cz = coordinates[:, :, 2]
    sx, sy, sz = _fps(cx, cy, cz)

    ssx = jnp.broadcast_to(sx.reshape(-1)[:, None], (B * SAMPLE_N, LANES)).reshape(-1)
    ssy = jnp.broadcast_to(sy.reshape(-1)[:, None], (B * SAMPLE_N, LANES)).reshape(-1)
    ssz = jnp.broadcast_to(sz.reshape(-1)[:, None], (B * SAMPLE_N, LANES)).reshape(-1)

    knn_idx = _knn(cx, cy, cz, ssx, ssy, ssz)
    idxt = (
        knn_idx.reshape(B, SAMPLE_N, KNN)
        .transpose(0, 2, 1)
        .reshape(B, KNN * SAMPLE_N)
    )
    pooled_t = _pool(x, idxt)
    return pooled_t.transpose(0, 2, 1).reshape(B, -1)


# speedup vs baseline: 2.7246x; 1.0111x over previous
import functools

import jax
import jax.numpy as jnp
from jax import lax
from jax.experimental import pallas as pl
from jax.experimental.pallas import tpu as pltpu
from jax.experimental.pallas import tpu_sc as plsc

SAMPLE_N = 512
KNN = 16
LANES = 16
NCORES = 2
NSUB = 16
NW = NCORES * NSUB

BATCH = 8
NPTS = 8192
NCH = 256

ROWS_W = (BATCH * SAMPLE_N) // NW
NCHUNK = NPTS // LANES
CAND_MAX = NPTS + LANES
W_PER_B = NW // BATCH
CH_W = NCH // W_PER_B
CH_BLK = 8


def _sc_mesh():
    return plsc.VectorSubcoreMesh(
        core_axis_name="c", subcore_axis_name="s",
        num_cores=NCORES, num_subcores=NSUB,
    )


def _fps_body(cx_ref, cy_ref, cz_ref, sx_ref, sy_ref, sz_ref):
    cx = cx_ref[...]
    cy = cy_ref[...]
    cz = cz_ref[...]
    B, N = cx.shape

    dx = cx - cx[:, 0:1]
    dy = cy - cy[:, 0:1]
    dz = cz - cz[:, 0:1]
    min_d = dx * dx + dy * dy + dz * dz

    lane_iota = jax.lax.broadcasted_iota(jnp.int32, (B, N), 1)
    col_iota = jax.lax.broadcasted_iota(jnp.int32, (B, SAMPLE_N), 1)

    sx0 = jnp.where(col_iota == 0, cx[:, 0:1], 0.0)
    sy0 = jnp.where(col_iota == 0, cy[:, 0:1], 0.0)
    sz0 = jnp.where(col_iota == 0, cz[:, 0:1], 0.0)

    def body(i, carry):
        min_d, sx, sy, sz = carry
        m = jnp.max(min_d, axis=1, keepdims=True)
        idx = jnp.min(
            jnp.where(min_d == m, lane_iota, jnp.int32(N)), axis=1, keepdims=True
        )
        sel = lane_iota == idx
        px = jnp.sum(jnp.where(sel, cx, 0.0), axis=1, keepdims=True)
        py = jnp.sum(jnp.where(sel, cy, 0.0), axis=1, keepdims=True)
        pz = jnp.sum(jnp.where(sel, cz, 0.0), axis=1, keepdims=True)
        ddx = cx - px
        ddy = cy - py
        ddz = cz - pz
        d = ddx * ddx + ddy * ddy + ddz * ddz
        min_d = jnp.minimum(min_d, d)
        upd = col_iota == i
        sx = jnp.where(upd, px, sx)
        sy = jnp.where(upd, py, sy)
        sz = jnp.where(upd, pz, sz)
        return (min_d, sx, sy, sz)

    _, sx, sy, sz = jax.lax.fori_loop(1, SAMPLE_N, body, (min_d, sx0, sy0, sz0))
    sx_ref[...] = sx
    sy_ref[...] = sy
    sz_ref[...] = sz


def _fps(cx, cy, cz):
    B = cx.shape[0]
    out = jax.ShapeDtypeStruct((B, SAMPLE_N), jnp.float32)
    return pl.pallas_call(_fps_body, out_shape=[out, out, out])(cx, cy, cz)


def _knn_body(cx_hbm, cy_hbm, cz_hbm, ssx_hbm, ssy_hbm, ssz_hbm, idx_hbm,
              cxv, cyv, czv, sxv, syv, szv, d2buf, candv, candi, idxout):
    wid = lax.axis_index("s") * NCORES + lax.axis_index("c")
    b = wid // W_PER_B
    r0 = wid * ROWS_W

    pltpu.sync_copy(cx_hbm.at[b], cxv)
    pltpu.sync_copy(cy_hbm.at[b], cyv)
    pltpu.sync_copy(cz_hbm.at[b], czv)
    pltpu.sync_copy(ssx_hbm.at[pl.ds(r0 * LANES, ROWS_W * LANES)], sxv)
    pltpu.sync_copy(ssy_hbm.at[pl.ds(r0 * LANES, ROWS_W * LANES)], syv)
    pltpu.sync_copy(ssz_hbm.at[pl.ds(r0 * LANES, ROWS_W * LANES)], szv)

    iota = lax.iota(jnp.int32, LANES)
    inf16 = jnp.full((LANES,), jnp.inf, jnp.float32)
    zero16i = jnp.zeros((LANES,), jnp.int32)

    def row_fn(r, _):
        sx = sxv[pl.ds(r * LANES, LANES)]
        sy = syv[pl.ds(r * LANES, LANES)]
        sz = szv[pl.ds(r * LANES, LANES)]

        @plsc.parallel_loop(0, NPTS, step=LANES, unroll=8, carry=inf16)
        def M(base, Mc):
            dx = sx - cxv[pl.ds(base, LANES)]
            dy = sy - cyv[pl.ds(base, LANES)]
            dz = sz - czv[pl.ds(base, LANES)]
            d2 = (dx * dx + dy * dy) + dz * dz
            d2buf[pl.ds(base, LANES)] = d2
            return jnp.minimum(Mc, d2)
        Mv, _ = plsc.sort_key_val(M, iota, descending=True)
        tau = Mv[0]

        def c2(j, cnt):
            base = j * (LANES * 8)
            offs = cnt
            for u in range(8):
                ub = base + u * LANES
                d2c = d2buf[pl.ds(ub, LANES)]
                m = d2c <= tau
                plsc.store_compressed(candv.at[pl.ds(offs, LANES)], d2c, mask=m)
                plsc.store_compressed(candi.at[pl.ds(offs, LANES)], iota + ub, mask=m)
                pc = plsc.all_reduce_population_count(m)
                offs = offs + pc[0]
            return offs

        cnt = lax.fori_loop(0, NCHUNK // 8, c2, jnp.int32(0))
        candv[pl.ds(cnt, LANES)] = inf16
        ng = (cnt + (LANES - 1)) // LANES

        def c3(g, carry):
            Rv, Ri = carry
            base = g * LANES
            Cv, Ci = plsc.sort_key_val(
                candv[pl.ds(base, LANES)], candi[pl.ds(base, LANES)],
                descending=True,
            )
            m = Cv < Rv
            Ev = jnp.where(m, Cv, Rv)
            Ei = jnp.where(m, Ci, Ri)
            Sv, Si = plsc.sort_key_val(Ev, Ei)
            return (Sv, Si)

        _, Ri = lax.fori_loop(0, ng, c3, (inf16, zero16i))
        idxout[pl.ds(r * LANES, LANES)] = Ri
        return 0

    lax.fori_loop(0, ROWS_W, row_fn, 0)
    pltpu.sync_copy(idxout, idx_hbm.at[pl.ds(r0 * LANES, ROWS_W * LANES)])


def _knn(cx, cy, cz, ssx, ssy, ssz):
    call = pl.kernel(
        _knn_body,
        out_type=jax.ShapeDtypeStruct((BATCH * SAMPLE_N * KNN,), jnp.int32),
        mesh=_sc_mesh(),
        compiler_params=pltpu.CompilerParams(needs_layout_passes=False),
        scratch_types=[
            pltpu.VMEM((NPTS,), jnp.float32),
            pltpu.VMEM((NPTS,), jnp.float32),
            pltpu.VMEM((NPTS,), jnp.float32),
            pltpu.VMEM((ROWS_W * LANES,), jnp.float32),
            pltpu.VMEM((ROWS_W * LANES,), jnp.float32),
            pltpu.VMEM((ROWS_W * LANES,), jnp.float32),
            pltpu.VMEM((NPTS,), jnp.float32),
            pltpu.VMEM((CAND_MAX,), jnp.float32),
            pltpu.VMEM((CAND_MAX,), jnp.int32),
            pltpu.VMEM((ROWS_W * LANES,), jnp.int32),
        ],
    )
    return call(cx, cy, cz, ssx, ssy, ssz)


def _pool_body(x_hbm, idxt_hbm, out_hbm, idxv, rowbuf, outbuf):
    wid = lax.axis_index("s") * NCORES + lax.axis_index("c")
    b = wid // W_PER_B
    c0 = (wid % W_PER_B) * CH_W

    pltpu.sync_copy(idxt_hbm.at[b], idxv)

    def cb_fn(cb, _):
        cbase = c0 + cb * CH_BLK
        pltpu.sync_copy(x_hbm.at[b, pl.ds(cbase, CH_BLK), :], rowbuf)

        def g_fn(g, _):
            s0 = g * LANES
            ivs = [idxv[pl.ds(k * SAMPLE_N + s0, LANES)] for k in range(KNN)]
            for cl in range(CH_BLK):
                row = jnp.full((LANES,), cl, jnp.int32)
                acc = plsc.load_gather(rowbuf, [row, ivs[0]])
                for k in range(1, KNN):
                    acc = jnp.maximum(acc, plsc.load_gather(rowbuf, [row, ivs[k]]))
                outbuf[cl, pl.ds(s0, LANES)] = acc
            return 0

        lax.fori_loop(0, SAMPLE_N // LANES, g_fn, 0)
        pltpu.sync_copy(outbuf, out_hbm.at[b, pl.ds(cbase, CH_BLK), :])
        return 0

    lax.fori_loop(0, CH_W // CH_BLK, cb_fn, 0)


def _pool(x, idxt):
    call = pl.kernel(
        _pool_body,
        out_type=jax.ShapeDtypeStruct((BATCH, NCH, SAMPLE_N), jnp.float32),
        mesh=_sc_mesh(),
        compiler_params=pltpu.CompilerParams(needs_layout_passes=False),
        scratch_types=[
            pltpu.VMEM((KNN * SAMPLE_N,), jnp.int32),
            pltpu.VMEM((CH_BLK, NPTS), jnp.float32),
            pltpu.VMEM((CH_BLK, SAMPLE_N), jnp.float32),
        ],
    )
    return call(x, idxt)


def kernel(x, coordinates):
    B, N, _ = coordinates.shape
    cx = coordinates[:, :, 0]
    cy = coordinates[:, :, 1]
    cz = coordinates[:, :, 2]
    sx, sy, sz = _fps(cx, cy, cz)

    ssx = jnp.broadcast_to(sx.reshape(-1)[:, None], (B * SAMPLE_N, LANES)).reshape(-1)
    ssy = jnp.broadcast_to(sy.reshape(-1)[:, None], (B * SAMPLE_N, LANES)).reshape(-1)
    ssz = jnp.broadcast_to(sz.reshape(-1)[:, None], (B * SAMPLE_N, LANES)).reshape(-1)

    knn_idx = _knn(cx, cy, cz, ssx, ssy, ssz)
    idxt = (
        knn_idx.reshape(B, SAMPLE_N, KNN)
        .transpose(0, 2, 1)
        .reshape(B, KNN * SAMPLE_N)
    )
    pooled_t = _pool(x, idxt)
    return pooled_t.transpose(0, 2, 1).reshape(B, -1)
